# baseline (speedup 1.0000x reference)
import jax
import jax.numpy as jnp
from jax import lax
from jax.experimental import pallas as pl
from jax.experimental.pallas import tpu as pltpu
from jax.experimental.pallas import tpu_sc as plsc

_NC, _NS = 2, 16
_NW = _NC * _NS
_CH = 128
_BC = 2048


def _sc_gather(user, item, eugT, eigT, eum, eim):
    B = user.shape[0]
    bpw = B // _NW
    nch = bpw // _CH
    nbc = B // _BC
    F, V = eugT.shape
    DM = eum.shape[1]
    f32 = jnp.float32
    mesh = plsc.VectorSubcoreMesh(core_axis_name="c", subcore_axis_name="s",
                                  num_cores=_NC, num_subcores=_NS)

    def body(user_h, item_h, eugT_h, eigT_h, eum_h, eim_h,
             prod_o, um_o, im_o,
             feat, valsc, idxa, idxb, mbuf, uidx, iidx, sems, semf, semi):
        wid = lax.axis_index("s") * _NC + lax.axis_index("c")
        base = wid * bpw
        idxc = (idxa, idxb)
        cpf = pltpu.async_copy(eugT_h.at[wid], feat, semf)
        pltpu.sync_copy(user_h.at[pl.ds(base, bpw)], uidx)
        pltpu.sync_copy(item_h.at[pl.ds(base, bpw)], iidx)

        mlp = ((eum_h, uidx, um_o), (eim_h, iidx, im_o))
        nst = 2 * nch

        def fire(j):
            th, ix, oo = mlp[j // nch]
            jj = j % nch
            cp = pltpu.async_copy(th.at[ix.at[pl.ds(jj * _CH, _CH)]],
                                  mbuf, sems)
            return (cp, oo, jj)

        def drain(cur, j):
            cp, oo, jj = cur
            cp.wait()
            pltpu.sync_copy(mbuf, oo.at[pl.ds(base + jj * _CH, _CH)])
            return fire(j + 1) if j + 1 < nst else None

        cur = fire(0)
        step = 0
        cpi = pltpu.async_copy(user_h.at[pl.ds(0, _BC)], idxc[0], semi)
        cpf.wait()
        for c in range(nbc):
            cpi.wait()
            if c < nbc - 1:
                cpi = pltpu.async_copy(
                    user_h.at[pl.ds((c + 1) * _BC, _BC)], idxc[(c + 1) % 2],
                    semi)
            else:
                cpi = pltpu.async_copy(item_h.at[pl.ds(0, _BC)], idxc[(c + 1) % 2],
                                       semi)
            cidx = idxc[c % 2]
            def ga(g, carry, cidx=cidx):
                for u in range(8):
                    vec = cidx[pl.ds((g * 8 + u) * 16, 16)]
                    valsc[pl.ds((g * 8 + u) * 16, 16)] = plsc.load_gather(
                        feat, [vec])
                return carry
            lax.fori_loop(0, _BC // 128, ga, 0)
            if c == nbc - 1:
                cpf = pltpu.async_copy(eigT_h.at[wid], feat, semf)
            pltpu.sync_copy(valsc, prod_o.at[wid, pl.ds(c * _BC, _BC)])
            cur = drain(cur, step)
            step += 1
        for c in range(nbc):
            cpi.wait()
            if c < nbc - 1:
                cpi = pltpu.async_copy(
                    item_h.at[pl.ds((c + 1) * _BC, _BC)],
                    idxc[(nbc + c + 1) % 2], semi)
            pltpu.sync_copy(prod_o.at[wid, pl.ds(c * _BC, _BC)], valsc)
            if c == 0:
                cpf.wait()
            cidx = idxc[(nbc + c) % 2]
            def gb(g, carry, cidx=cidx):
                for u in range(8):
                    sl = pl.ds((g * 8 + u) * 16, 16)
                    valsc[sl] = valsc[sl] * plsc.load_gather(
                        feat, [cidx[sl]])
                return carry
            lax.fori_loop(0, _BC // 128, gb, 0)
            pltpu.sync_copy(valsc, prod_o.at[wid, pl.ds(c * _BC, _BC)])
            if cur is not None:
                cur = drain(cur, step)
                step += 1

    k = pl.kernel(
        body,
        out_type=(jax.ShapeDtypeStruct((F, B), f32),
                  jax.ShapeDtypeStruct((B, DM), f32),
                  jax.ShapeDtypeStruct((B, DM), f32)),
        mesh=mesh,
        compiler_params=pltpu.CompilerParams(needs_layout_passes=False),
        scratch_types=[
            pltpu.VMEM((V,), f32),
            pltpu.VMEM((_BC,), f32),
            pltpu.VMEM((_BC,), jnp.int32),
            pltpu.VMEM((_BC,), jnp.int32),
            pltpu.VMEM((_CH, DM), f32),
            pltpu.VMEM((bpw,), jnp.int32),
            pltpu.VMEM((bpw,), jnp.int32),
            pltpu.SemaphoreType.DMA,
            pltpu.SemaphoreType.DMA,
            pltpu.SemaphoreType.DMA,
        ],
    )
    return k(user, item, eugT, eigT, eum, eim)


def _tc_mlp(prod, um, im, W0a, W0b, b0, W1, b1, W2, b2, wpgT, wph, bp):
    F, B = prod.shape
    DM = um.shape[1]
    BT = 4096
    f32 = jnp.float32

    def body(prod_r, um_r, im_r, W0a_r, W0b_r, b0_r, W1_r, b1_r,
             W2_r, b2_r, wpgT_r, wph_r, bp_r, out_r):
        h = jnp.dot(um_r[...], W0a_r[...], preferred_element_type=f32)
        h = h + jnp.dot(im_r[...], W0b_r[...], preferred_element_type=f32)
        h = jnp.maximum(h + b0_r[...], 0.0)
        h = jnp.maximum(
            jnp.dot(h, W1_r[...], preferred_element_type=f32) + b1_r[...], 0.0)
        h = jnp.maximum(
            jnp.dot(h, W2_r[...], preferred_element_type=f32) + b2_r[...], 0.0)
        p = (jnp.sum(prod_r[...] * wpgT_r[...], axis=0)
             + jnp.sum(h * wph_r[...], axis=1) + bp_r[0])
        out_r[...] = p

    full = lambda shape: pl.BlockSpec(shape, lambda i: tuple(0 for _ in shape))
    out = pl.pallas_call(
        body,
        grid=(B // BT,),
        in_specs=[
            pl.BlockSpec((F, BT), lambda i: (0, i)),
            pl.BlockSpec((BT, DM), lambda i: (i, 0)),
            pl.BlockSpec((BT, DM), lambda i: (i, 0)),
            full((DM, DM)), full((DM, DM)), full((1, DM)),
            full((DM, DM // 2)), full((1, DM // 2)),
            full((DM // 2, DM // 4)), full((1, DM // 4)),
            full((F, 1)), full((1, F)),
            pl.BlockSpec(memory_space=pltpu.SMEM),
        ],
        out_specs=pl.BlockSpec((BT,), lambda i: (i,)),
        out_shape=jax.ShapeDtypeStruct((B,), f32),
    )(prod, um, im, W0a, W0b, b0, W1, b1, W2, b2, wpgT, wph, bp)
    return out


def kernel(user, item, emb_user_gmf, emb_item_gmf, emb_user_mlp, emb_item_mlp,
           W0, b0, W1, b1, W2, b2, Wp, bp):
    F = emb_user_gmf.shape[1]
    DM = emb_user_mlp.shape[1]
    user = user.astype(jnp.int32)
    item = item.astype(jnp.int32)
    prod, um, im = _sc_gather(user, item, emb_user_gmf.T, emb_item_gmf.T,
                              emb_user_mlp, emb_item_mlp)
    return _tc_mlp(prod, um, im,
                   W0[:DM], W0[DM:], b0.reshape(1, DM),
                   W1, b1.reshape(1, DM // 2),
                   W2, b2.reshape(1, DM // 4),
                   Wp[:F].reshape(F, 1), Wp[F:].reshape(1, F),
                   bp)


# speedup vs baseline: 1.0457x; 1.0457x over previous
import jax
import jax.numpy as jnp
from jax import lax
from jax.experimental import pallas as pl
from jax.experimental.pallas import tpu as pltpu
from jax.experimental.pallas import tpu_sc as plsc

_NC, _NS = 2, 16
_NW = _NC * _NS
_CH = 128
_BC = 4096


def _sc_gather(user, item, eugT, eigT, eum, eim):
    B = user.shape[0]
    bpw = B // _NW
    nch = bpw // _CH
    nbc = B // _BC
    F, V = eugT.shape
    DM = eum.shape[1]
    f32 = jnp.float32
    mesh = plsc.VectorSubcoreMesh(core_axis_name="c", subcore_axis_name="s",
                                  num_cores=_NC, num_subcores=_NS)

    def body(user_h, item_h, eugT_h, eigT_h, eum_h, eim_h,
             prod_o, um_o, im_o,
             feat, valsc, idxc, mbuf, uidx, iidx, sems, semf):
        wid = lax.axis_index("s") * _NC + lax.axis_index("c")
        base = wid * bpw
        cpf = pltpu.async_copy(eugT_h.at[wid], feat, semf)
        pltpu.sync_copy(user_h.at[pl.ds(base, bpw)], uidx)
        pltpu.sync_copy(item_h.at[pl.ds(base, bpw)], iidx)

        mlp = ((eum_h, uidx, um_o), (eim_h, iidx, im_o))
        nst = 2 * nch

        def fire(j):
            th, ix, oo = mlp[j // nch]
            jj = j % nch
            cp = pltpu.async_copy(th.at[ix.at[pl.ds(jj * _CH, _CH)]],
                                  mbuf, sems)
            return (cp, oo, jj)

        def drain(cur, j):
            cp, oo, jj = cur
            cp.wait()
            pltpu.sync_copy(mbuf, oo.at[pl.ds(base + jj * _CH, _CH)])
            return fire(j + 1) if j + 1 < nst else None

        cur = fire(0)
        step = 0
        cpf.wait()
        for c in range(nbc):
            pltpu.sync_copy(user_h.at[pl.ds(c * _BC, _BC)], idxc)
            def ga(g, carry):
                for u in range(8):
                    vec = idxc[pl.ds((g * 8 + u) * 16, 16)]
                    valsc[pl.ds((g * 8 + u) * 16, 16)] = plsc.load_gather(
                        feat, [vec])
                return carry
            lax.fori_loop(0, _BC // 128, ga, 0)
            if c == nbc - 1:
                cpf = pltpu.async_copy(eigT_h.at[wid], feat, semf)
            pltpu.sync_copy(valsc, prod_o.at[wid, pl.ds(c * _BC, _BC)])
            cur = drain(cur, step)
            step += 1
        for c in range(nbc):
            pltpu.sync_copy(item_h.at[pl.ds(c * _BC, _BC)], idxc)
            pltpu.sync_copy(prod_o.at[wid, pl.ds(c * _BC, _BC)], valsc)
            if c == 0:
                cpf.wait()
            def gb(g, carry):
                for u in range(8):
                    sl = pl.ds((g * 8 + u) * 16, 16)
                    valsc[sl] = valsc[sl] * plsc.load_gather(
                        feat, [idxc[sl]])
                return carry
            lax.fori_loop(0, _BC // 128, gb, 0)
            pltpu.sync_copy(valsc, prod_o.at[wid, pl.ds(c * _BC, _BC)])
            if cur is not None:
                cur = drain(cur, step)
                step += 1

    k = pl.kernel(
        body,
        out_type=(jax.ShapeDtypeStruct((F, B), f32),
                  jax.ShapeDtypeStruct((B, DM), f32),
                  jax.ShapeDtypeStruct((B, DM), f32)),
        mesh=mesh,
        compiler_params=pltpu.CompilerParams(needs_layout_passes=False),
        scratch_types=[
            pltpu.VMEM((V,), f32),
            pltpu.VMEM((_BC,), f32),
            pltpu.VMEM((_BC,), jnp.int32),
            pltpu.VMEM((_CH, DM), f32),
            pltpu.VMEM((bpw,), jnp.int32),
            pltpu.VMEM((bpw,), jnp.int32),
            pltpu.SemaphoreType.DMA,
            pltpu.SemaphoreType.DMA,
        ],
    )
    return k(user, item, eugT, eigT, eum, eim)


def _tc_mlp(prod, um, im, W0a, W0b, b0, W1, b1, W2, b2, wpgT, wph, bp):
    F, B = prod.shape
    DM = um.shape[1]
    BT = 4096
    f32 = jnp.float32

    def body(prod_r, um_r, im_r, W0a_r, W0b_r, b0_r, W1_r, b1_r,
             W2_r, b2_r, wpgT_r, wph_r, bp_r, out_r):
        h = jnp.dot(um_r[...], W0a_r[...], preferred_element_type=f32)
        h = h + jnp.dot(im_r[...], W0b_r[...], preferred_element_type=f32)
        h = jnp.maximum(h + b0_r[...], 0.0)
        h = jnp.maximum(
            jnp.dot(h, W1_r[...], preferred_element_type=f32) + b1_r[...], 0.0)
        h = jnp.maximum(
            jnp.dot(h, W2_r[...], preferred_element_type=f32) + b2_r[...], 0.0)
        p = (jnp.sum(prod_r[...] * wpgT_r[...], axis=0)
             + jnp.sum(h * wph_r[...], axis=1) + bp_r[0])
        out_r[...] = p

    full = lambda shape: pl.BlockSpec(shape, lambda i: tuple(0 for _ in shape))
    out = pl.pallas_call(
        body,
        grid=(B // BT,),
        in_specs=[
            pl.BlockSpec((F, BT), lambda i: (0, i)),
            pl.BlockSpec((BT, DM), lambda i: (i, 0)),
            pl.BlockSpec((BT, DM), lambda i: (i, 0)),
            full((DM, DM)), full((DM, DM)), full((1, DM)),
            full((DM, DM // 2)), full((1, DM // 2)),
            full((DM // 2, DM // 4)), full((1, DM // 4)),
            full((F, 1)), full((1, F)),
            pl.BlockSpec(memory_space=pltpu.SMEM),
        ],
        out_specs=pl.BlockSpec((BT,), lambda i: (i,)),
        out_shape=jax.ShapeDtypeStruct((B,), f32),
    )(prod, um, im, W0a, W0b, b0, W1, b1, W2, b2, wpgT, wph, bp)
    return out


def kernel(user, item, emb_user_gmf, emb_item_gmf, emb_user_mlp, emb_item_mlp,
           W0, b0, W1, b1, W2, b2, Wp, bp):
    F = emb_user_gmf.shape[1]
    DM = emb_user_mlp.shape[1]
    user = user.astype(jnp.int32)
    item = item.astype(jnp.int32)
    prod, um, im = _sc_gather(user, item, emb_user_gmf.T, emb_item_gmf.T,
                              emb_user_mlp, emb_item_mlp)
    return _tc_mlp(prod, um, im,
                   W0[:DM], W0[DM:], b0.reshape(1, DM),
                   W1, b1.reshape(1, DM // 2),
                   W2, b2.reshape(1, DM // 4),
                   Wp[:F].reshape(F, 1), Wp[F:].reshape(1, F),
                   bp)
